# baseline (speedup 1.0000x reference)
import functools

import jax
import jax.numpy as jnp
from jax import lax
from jax.experimental import pallas as pl
from jax.experimental.pallas import tpu as pltpu
from jax.experimental.pallas import tpu_sc as plsc

DIM = 16
BATCH = 16384
NUM_CORES = 2
NUM_SUBCORES = 16
NUM_WORKERS = NUM_CORES * NUM_SUBCORES
ROWS_PER_WORKER = BATCH // NUM_WORKERS
CHUNK = 128
CHUNKS_PER_WORKER = ROWS_PER_WORKER // CHUNK
GROUPS = ROWS_PER_WORKER // 16


def _mfbias_body(ui_hbm, ii_hbm, ut_hbm, it_hbm, ub_hbm, ib_hbm, gb_hbm,
                 out_hbm,
                 uidx_v, iidx_v, urows_v, irows_v, ub_v, ib_v, gb_v,
                 prod_v, out_v, sem):
    wid = lax.axis_index("s") * NUM_CORES + lax.axis_index("c")
    crow0 = wid * CHUNKS_PER_WORKER

    pltpu.sync_copy(ui_hbm.at[pl.ds(crow0, CHUNKS_PER_WORKER)], uidx_v)
    pltpu.sync_copy(ii_hbm.at[pl.ds(crow0, CHUNKS_PER_WORKER)], iidx_v)
    pltpu.sync_copy(gb_hbm, gb_v)

    handles = []
    for j in range(CHUNKS_PER_WORKER):
        dst = pl.ds(j * CHUNK, CHUNK)
        handles.append(pltpu.async_copy(
            ut_hbm.at[uidx_v.at[j]], urows_v.at[dst], sem))
        handles.append(pltpu.async_copy(
            it_hbm.at[iidx_v.at[j]], irows_v.at[dst], sem))
        handles.append(pltpu.async_copy(
            ub_hbm.at[uidx_v.at[j]], ub_v.at[dst], sem))
        handles.append(pltpu.async_copy(
            ib_hbm.at[iidx_v.at[j]], ib_v.at[dst], sem))
    for h in handles:
        h.wait()

    gb = gb_v[...]
    fbase = lax.iota(jnp.int32, 16) * DIM

    def group(g, carry):
        r0 = g * 16
        for k in range(16):
            r = r0 + k
            prod_v[pl.ds(k * DIM, DIM)] = urows_v[r, :] * irows_v[r, :]
        acc = ub_v[pl.ds(r0, 16)] + ib_v[pl.ds(r0, 16)] + gb
        for d in range(DIM):
            acc = acc + plsc.load_gather(prod_v, [fbase + d])
        out_v[pl.ds(r0, 16)] = acc
        return carry

    lax.fori_loop(0, GROUPS, group, 0)
    pltpu.sync_copy(out_v, out_hbm.at[pl.ds(wid * ROWS_PER_WORKER,
                                            ROWS_PER_WORKER)])


@functools.partial(jax.jit)
def _mfbias_call(ui2, ii2, user_table, item_table, user_bias, item_bias,
                 gb16):
    mesh = plsc.VectorSubcoreMesh(core_axis_name="c", subcore_axis_name="s")
    run = pl.kernel(
        _mfbias_body,
        out_type=jax.ShapeDtypeStruct((BATCH,), jnp.float32),
        mesh=mesh,
        compiler_params=pltpu.CompilerParams(
            needs_layout_passes=False, use_tc_tiling_on_sc=False),
        scratch_types=[
            pltpu.VMEM((CHUNKS_PER_WORKER, CHUNK), jnp.int32),
            pltpu.VMEM((CHUNKS_PER_WORKER, CHUNK), jnp.int32),
            pltpu.VMEM((ROWS_PER_WORKER, DIM), jnp.float32),
            pltpu.VMEM((ROWS_PER_WORKER, DIM), jnp.float32),
            pltpu.VMEM((ROWS_PER_WORKER,), jnp.float32),
            pltpu.VMEM((ROWS_PER_WORKER,), jnp.float32),
            pltpu.VMEM((16,), jnp.float32),
            pltpu.VMEM((16 * DIM,), jnp.float32),
            pltpu.VMEM((ROWS_PER_WORKER,), jnp.float32),
            pltpu.SemaphoreType.DMA,
        ],
    )
    return run(ui2, ii2, user_table, item_table, user_bias, item_bias, gb16)


def kernel(user_indices, item_indices, user_table, item_table, user_bias,
           item_bias, global_bias):
    ui2 = user_indices.astype(jnp.int32).reshape(
        NUM_WORKERS * CHUNKS_PER_WORKER, CHUNK)
    ii2 = item_indices.astype(jnp.int32).reshape(
        NUM_WORKERS * CHUNKS_PER_WORKER, CHUNK)
    gb16 = jnp.broadcast_to(global_bias.astype(jnp.float32), (16,))
    return _mfbias_call(ui2, ii2, user_table, item_table,
                        user_bias, item_bias, gb16)
